# baseline (speedup 1.0000x reference)
import jax
import jax.numpy as jnp
from jax.experimental import pallas as pl
from jax.experimental.pallas import tpu as pltpu

_B, _NF, _NLF, _NFEAT, _NCLASS, _HID, _E, _TOPK = 1024, 6, 3, 256, 349, 256, 8, 2
_C = _NF + _NLF
_TB = 256


def _spectral_inv(w):
    g0 = jax.lax.dot_general(w, w, (((1,), (1,)), ((), ())),
                             preferred_element_type=jnp.float32)

    def body(_, g):
        g = jnp.dot(g, g, preferred_element_type=jnp.float32)
        return g * (1.0 / jnp.max(jnp.abs(g)))

    g = jax.lax.fori_loop(0, 8, body, g0 * (1.0 / jnp.max(jnp.abs(g0))))
    v = jnp.sum(g, axis=1, keepdims=True)
    gv = jnp.dot(g0, v, preferred_element_type=jnp.float32)
    lam = jnp.sum(v * gv) / jnp.sum(v * v)
    return jax.lax.rsqrt(lam)


def _sigma_kernel(wq_ref, wk_ref, wv_ref, out_ref):
    out_ref[0, 0] = _spectral_inv(wq_ref[...]) * _spectral_inv(wk_ref[...])
    out_ref[0, 1] = _spectral_inv(wv_ref[...])


def _proj_attn_kernel(inv_ref, x_ref, lf_ref, W1_ref, b1_ref, W2_ref, b2_ref,
                      Wl1_ref, bl1_ref, Wl2_ref, bl2_ref,
                      g1_ref, be1_ref, g2_ref, be2_ref,
                      gl1_ref, bel1_ref, gl2_ref, bel2_ref,
                      Wq_ref, Wk_ref, Wv_ref, Wg_ref, bg_ref,
                      a1_ref, a2_ref, al1_ref, al2_ref, gamma_ref,
                      flat_ref, w_ref):
    inv_qk = inv_ref[0, 0]
    inv_v = inv_ref[0, 1]
    gamma = gamma_ref[0]

    def proj_layer(zs, W_ref, b_ref, g_ref, beta_ref, a, nc):
        hs = [jnp.dot(zs[c], W_ref[c], preferred_element_type=jnp.float32)
              + b_ref[c:c + 1, :] for c in range(nc)]
        n = nc * _HID
        tot = sum(jnp.sum(h, axis=1, keepdims=True) for h in hs)
        mean = tot * (1.0 / n)
        tot2 = sum(jnp.sum(h * h, axis=1, keepdims=True) for h in hs)
        var = tot2 * (1.0 / n) - mean * mean
        inv = jax.lax.rsqrt(var + 1e-5)
        outs = []
        for c in range(nc):
            o = (hs[c] - mean) * inv * g_ref[c:c + 1, :] + beta_ref[c:c + 1, :]
            outs.append(jnp.where(o > 0, o, a * o))
        return outs

    xs = [x_ref[:, c, :] for c in range(_NF)]
    hs = proj_layer(xs, W1_ref, b1_ref, g1_ref, be1_ref, a1_ref[0], _NF)
    hs = proj_layer(hs, W2_ref, b2_ref, g2_ref, be2_ref, a2_ref[0], _NF)
    lfs = [lf_ref[:, c, :] for c in range(_NLF)]
    hls = proj_layer(lfs, Wl1_ref, bl1_ref, gl1_ref, bel1_ref, al1_ref[0], _NLF)
    hls = proj_layer(hls, Wl2_ref, bl2_ref, gl2_ref, bel2_ref, al2_ref[0], _NLF)
    zs = hs + hls

    dims = (((1,), (1,)), ((), ()))
    Fs = [jax.lax.dot_general(z, Wq_ref[...], dims,
                              preferred_element_type=jnp.float32) for z in zs]
    Gs = [jax.lax.dot_general(z, Wk_ref[...], dims,
                              preferred_element_type=jnp.float32) for z in zs]
    Vs = [jax.lax.dot_general(z, Wv_ref[...], dims,
                              preferred_element_type=jnp.float32) for z in zs]

    logits = bg_ref[0:1, :]
    for d in range(_C):
        col = jnp.concatenate(
            [jnp.sum(Fs[c] * Gs[d], axis=1, keepdims=True) for c in range(_C)],
            axis=1) * inv_qk
        col = jnp.maximum(col, 0.0)
        m = jnp.max(col, axis=1, keepdims=True)
        e = jnp.exp(col - m)
        beta_d = e * (1.0 / jnp.sum(e, axis=1, keepdims=True))
        acc = jnp.zeros_like(Vs[0])
        for c in range(_C):
            acc = acc + beta_d[:, c:c + 1] * Vs[c]
        o_z = gamma * inv_v * acc + zs[d]
        flat_ref[:, d * _HID:(d + 1) * _HID] = o_z
        logits = logits + jnp.dot(o_z, Wg_ref[d],
                                  preferred_element_type=jnp.float32)

    idx = jax.lax.broadcasted_iota(jnp.int32, logits.shape, 1)
    m1 = jnp.max(logits, axis=1, keepdims=True)
    i1 = jnp.min(jnp.where(logits == m1, idx, _E), axis=1, keepdims=True)
    masked = jnp.where(idx == i1, -jnp.inf, logits)
    m2 = jnp.max(masked, axis=1, keepdims=True)
    i2 = jnp.min(jnp.where(masked == m2, idx, _E), axis=1, keepdims=True)
    s1 = 1.0 / (1.0 + jnp.exp(m2 - m1))
    s2 = 1.0 - s1
    w_ref[...] = s1 * (idx == i1).astype(jnp.float32) \
        + s2 * (idx == i2).astype(jnp.float32)


def _moe_kernel(flat_ref, w_ref, We_ref, be_ref, out_ref):
    e = pl.program_id(1)

    @pl.when(e == 0)
    def _():
        out_ref[...] = jnp.zeros_like(out_ref)

    idx = jax.lax.broadcasted_iota(jnp.int32, w_ref.shape, 1)
    w_e = jnp.sum(jnp.where(idx == e, w_ref[...], 0.0), axis=1, keepdims=True)
    exp_out = jnp.dot(flat_ref[...], We_ref[0],
                      preferred_element_type=jnp.float32) + be_ref[0]
    out_ref[...] = out_ref[...] + w_e * exp_out


def kernel(x, label_feats, W1, b1, W2, b2, Wl1, bl1, Wl2, bl2,
           ln1_g, ln1_b, ln2_g, ln2_b, lnl1_g, lnl1_b, lnl2_g, lnl2_b,
           a1, a2, al1, al2, Wq, Wk, Wv, gamma, Wg, bg, We, be):
    Wg_r = Wg.reshape(_C, _HID, _E)
    bg_r = bg.reshape(1, _E)

    nblk = _B // _TB
    full = lambda arr: pl.BlockSpec(arr.shape, lambda i: (0,) * arr.ndim)
    smem = pl.BlockSpec(memory_space=pltpu.SMEM)

    inv = pl.pallas_call(
        _sigma_kernel,
        out_shape=jax.ShapeDtypeStruct((1, 2), jnp.float32),
        out_specs=pl.BlockSpec(memory_space=pltpu.SMEM),
    )(Wq, Wk, Wv)

    flat, w = pl.pallas_call(
        _proj_attn_kernel,
        grid=(nblk,),
        in_specs=[
            smem,
            pl.BlockSpec((_TB, _NF, _NFEAT), lambda i: (i, 0, 0)),
            pl.BlockSpec((_TB, _NLF, _NCLASS), lambda i: (i, 0, 0)),
            full(W1), full(b1), full(W2), full(b2),
            full(Wl1), full(bl1), full(Wl2), full(bl2),
            full(ln1_g), full(ln1_b), full(ln2_g), full(ln2_b),
            full(lnl1_g), full(lnl1_b), full(lnl2_g), full(lnl2_b),
            full(Wq), full(Wk), full(Wv), full(Wg_r), full(bg_r),
            smem, smem, smem, smem, smem,
        ],
        out_specs=[
            pl.BlockSpec((_TB, _C * _HID), lambda i: (i, 0)),
            pl.BlockSpec((_TB, _E), lambda i: (i, 0)),
        ],
        out_shape=[
            jax.ShapeDtypeStruct((_B, _C * _HID), jnp.float32),
            jax.ShapeDtypeStruct((_B, _E), jnp.float32),
        ],
        compiler_params=pltpu.CompilerParams(
            dimension_semantics=("parallel",),
        ),
    )(inv, x, label_feats, W1, b1, W2, b2, Wl1, bl1, Wl2, bl2,
      ln1_g, ln1_b, ln2_g, ln2_b, lnl1_g, lnl1_b, lnl2_g, lnl2_b,
      Wq, Wk, Wv, Wg_r, bg_r, a1, a2, al1, al2, gamma)

    hb = _B // 2
    out = pl.pallas_call(
        _moe_kernel,
        grid=(2, _E),
        in_specs=[
            pl.BlockSpec((hb, _C * _HID), lambda i, e: (i, 0)),
            pl.BlockSpec((hb, _E), lambda i, e: (i, 0)),
            pl.BlockSpec((1, _C * _HID, _HID), lambda i, e: (e, 0, 0)),
            pl.BlockSpec((1, 1, _HID), lambda i, e: (e, 0, 0)),
        ],
        out_specs=pl.BlockSpec((hb, _HID), lambda i, e: (i, 0)),
        out_shape=jax.ShapeDtypeStruct((_B, _HID), jnp.float32),
        compiler_params=pltpu.CompilerParams(
            dimension_semantics=("parallel", "arbitrary"),
        ),
    )(flat, w, We, be.reshape(_E, 1, _HID))
    return out


# speedup vs baseline: 1.0753x; 1.0753x over previous
import jax
import jax.numpy as jnp
from jax.experimental import pallas as pl
from jax.experimental.pallas import tpu as pltpu

_B, _NF, _NLF, _NFEAT, _NCLASS, _HID, _E, _TOPK = 1024, 6, 3, 256, 349, 256, 8, 2
_C = _NF + _NLF
_TB = 256


def _spectral_inv(w):
    g0 = jax.lax.dot_general(w, w, (((1,), (1,)), ((), ())),
                             preferred_element_type=jnp.float32)

    def body(_, g):
        g = jnp.dot(g, g, preferred_element_type=jnp.float32)
        return g * (1.0 / jnp.max(jnp.abs(g)))

    g = jax.lax.fori_loop(0, 8, body, g0 * (1.0 / jnp.max(jnp.abs(g0))))
    v = jnp.sum(g, axis=1, keepdims=True)
    gv = jnp.dot(g0, v, preferred_element_type=jnp.float32)
    lam = jnp.sum(v * gv) / jnp.sum(v * v)
    return jax.lax.rsqrt(lam)


def _proj_attn_kernel(x_ref, lf_ref, W1_ref, b1_ref, W2_ref, b2_ref,
                      Wl1_ref, bl1_ref, Wl2_ref, bl2_ref,
                      g1_ref, be1_ref, g2_ref, be2_ref,
                      gl1_ref, bel1_ref, gl2_ref, bel2_ref,
                      Wq_ref, Wk_ref, Wv_ref, Wg_ref, bg_ref,
                      a1_ref, a2_ref, al1_ref, al2_ref, gamma_ref,
                      flat_ref, w_ref, inv_scr):
    i = pl.program_id(0)

    @pl.when(i == 0)
    def _():
        inv_scr[0] = _spectral_inv(Wq_ref[...]) * _spectral_inv(Wk_ref[...])
        inv_scr[1] = _spectral_inv(Wv_ref[...])

    inv_qk = inv_scr[0]
    inv_v = inv_scr[1]
    gamma = gamma_ref[0]

    def proj_layer(zs, W_ref, b_ref, g_ref, beta_ref, a, nc):
        hs = [jnp.dot(zs[c], W_ref[c], preferred_element_type=jnp.float32)
              + b_ref[c:c + 1, :] for c in range(nc)]
        n = nc * _HID
        tot = sum(jnp.sum(h, axis=1, keepdims=True) for h in hs)
        mean = tot * (1.0 / n)
        tot2 = sum(jnp.sum(h * h, axis=1, keepdims=True) for h in hs)
        var = tot2 * (1.0 / n) - mean * mean
        inv = jax.lax.rsqrt(var + 1e-5)
        outs = []
        for c in range(nc):
            o = (hs[c] - mean) * inv * g_ref[c:c + 1, :] + beta_ref[c:c + 1, :]
            outs.append(jnp.where(o > 0, o, a * o))
        return outs

    xs = [x_ref[:, c, :] for c in range(_NF)]
    hs = proj_layer(xs, W1_ref, b1_ref, g1_ref, be1_ref, a1_ref[0], _NF)
    hs = proj_layer(hs, W2_ref, b2_ref, g2_ref, be2_ref, a2_ref[0], _NF)
    lfs = [lf_ref[:, c, :] for c in range(_NLF)]
    hls = proj_layer(lfs, Wl1_ref, bl1_ref, gl1_ref, bel1_ref, al1_ref[0], _NLF)
    hls = proj_layer(hls, Wl2_ref, bl2_ref, gl2_ref, bel2_ref, al2_ref[0], _NLF)
    zs = hs + hls

    dims = (((1,), (1,)), ((), ()))
    Fs = [jax.lax.dot_general(z, Wq_ref[...], dims,
                              preferred_element_type=jnp.float32) for z in zs]
    Gs = [jax.lax.dot_general(z, Wk_ref[...], dims,
                              preferred_element_type=jnp.float32) for z in zs]
    Vs = [jax.lax.dot_general(z, Wv_ref[...], dims,
                              preferred_element_type=jnp.float32) for z in zs]

    logits = bg_ref[0:1, :]
    for d in range(_C):
        col = jnp.concatenate(
            [jnp.sum(Fs[c] * Gs[d], axis=1, keepdims=True) for c in range(_C)],
            axis=1) * inv_qk
        col = jnp.maximum(col, 0.0)
        m = jnp.max(col, axis=1, keepdims=True)
        e = jnp.exp(col - m)
        beta_d = e * (1.0 / jnp.sum(e, axis=1, keepdims=True))
        acc = jnp.zeros_like(Vs[0])
        for c in range(_C):
            acc = acc + beta_d[:, c:c + 1] * Vs[c]
        o_z = gamma * inv_v * acc + zs[d]
        flat_ref[:, d * _HID:(d + 1) * _HID] = o_z
        logits = logits + jnp.dot(o_z, Wg_ref[d],
                                  preferred_element_type=jnp.float32)

    idx = jax.lax.broadcasted_iota(jnp.int32, logits.shape, 1)
    m1 = jnp.max(logits, axis=1, keepdims=True)
    i1 = jnp.min(jnp.where(logits == m1, idx, _E), axis=1, keepdims=True)
    masked = jnp.where(idx == i1, -jnp.inf, logits)
    m2 = jnp.max(masked, axis=1, keepdims=True)
    i2 = jnp.min(jnp.where(masked == m2, idx, _E), axis=1, keepdims=True)
    s1 = 1.0 / (1.0 + jnp.exp(m2 - m1))
    s2 = 1.0 - s1
    w_ref[...] = s1 * (idx == i1).astype(jnp.float32) \
        + s2 * (idx == i2).astype(jnp.float32)


def _moe_kernel(flat_ref, w_ref, We_ref, be_ref, out_ref):
    e = pl.program_id(1)

    @pl.when(e == 0)
    def _():
        out_ref[...] = jnp.zeros_like(out_ref)

    idx = jax.lax.broadcasted_iota(jnp.int32, w_ref.shape, 1)
    w_e = jnp.sum(jnp.where(idx == e, w_ref[...], 0.0), axis=1, keepdims=True)
    exp_out = jnp.dot(flat_ref[...], We_ref[0],
                      preferred_element_type=jnp.float32) + be_ref[0]
    out_ref[...] = out_ref[...] + w_e * exp_out


def kernel(x, label_feats, W1, b1, W2, b2, Wl1, bl1, Wl2, bl2,
           ln1_g, ln1_b, ln2_g, ln2_b, lnl1_g, lnl1_b, lnl2_g, lnl2_b,
           a1, a2, al1, al2, Wq, Wk, Wv, gamma, Wg, bg, We, be):
    Wg_r = Wg.reshape(_C, _HID, _E)
    bg_r = bg.reshape(1, _E)

    nblk = _B // _TB
    full = lambda arr: pl.BlockSpec(arr.shape, lambda i: (0,) * arr.ndim)
    smem = pl.BlockSpec(memory_space=pltpu.SMEM)

    flat, w = pl.pallas_call(
        _proj_attn_kernel,
        grid=(nblk,),
        in_specs=[
            pl.BlockSpec((_TB, _NF, _NFEAT), lambda i: (i, 0, 0)),
            pl.BlockSpec((_TB, _NLF, _NCLASS), lambda i: (i, 0, 0)),
            full(W1), full(b1), full(W2), full(b2),
            full(Wl1), full(bl1), full(Wl2), full(bl2),
            full(ln1_g), full(ln1_b), full(ln2_g), full(ln2_b),
            full(lnl1_g), full(lnl1_b), full(lnl2_g), full(lnl2_b),
            full(Wq), full(Wk), full(Wv), full(Wg_r), full(bg_r),
            smem, smem, smem, smem, smem,
        ],
        out_specs=[
            pl.BlockSpec((_TB, _C * _HID), lambda i: (i, 0)),
            pl.BlockSpec((_TB, _E), lambda i: (i, 0)),
        ],
        out_shape=[
            jax.ShapeDtypeStruct((_B, _C * _HID), jnp.float32),
            jax.ShapeDtypeStruct((_B, _E), jnp.float32),
        ],
        scratch_shapes=[pltpu.SMEM((2,), jnp.float32)],
        compiler_params=pltpu.CompilerParams(
            dimension_semantics=("arbitrary",),
        ),
    )(x, label_feats, W1, b1, W2, b2, Wl1, bl1, Wl2, bl2,
      ln1_g, ln1_b, ln2_g, ln2_b, lnl1_g, lnl1_b, lnl2_g, lnl2_b,
      Wq, Wk, Wv, Wg_r, bg_r, a1, a2, al1, al2, gamma)

    out = pl.pallas_call(
        _moe_kernel,
        grid=(1, _E),
        in_specs=[
            pl.BlockSpec((_B, _C * _HID), lambda i, e: (0, 0)),
            pl.BlockSpec((_B, _E), lambda i, e: (0, 0)),
            pl.BlockSpec((1, _C * _HID, _HID), lambda i, e: (e, 0, 0)),
            pl.BlockSpec((1, 1, _HID), lambda i, e: (e, 0, 0)),
        ],
        out_specs=pl.BlockSpec((_B, _HID), lambda i, e: (0, 0)),
        out_shape=jax.ShapeDtypeStruct((_B, _HID), jnp.float32),
        compiler_params=pltpu.CompilerParams(
            dimension_semantics=("arbitrary", "arbitrary"),
        ),
    )(flat, w, We, be.reshape(_E, 1, _HID))
    return out
